# baseline (speedup 1.0000x reference)
import functools
import math

import jax
import jax.numpy as jnp
from jax import lax
from jax.experimental import pallas as pl
from jax.experimental.pallas import tpu as pltpu
from jax.experimental.pallas import tpu_sc as plsc

D_MODEL = 64
ROWS = 4096
COLS = 200
B = ROWS * COLS
NUM_CORES = 2
NUM_SUBCORES = 16
NW = NUM_CORES * NUM_SUBCORES
CHUNK = 128
NCHUNK = B // (NW * CHUNK)
SCALE = math.sqrt(D_MODEL)

_mesh = plsc.VectorSubcoreMesh(core_axis_name="c", subcore_axis_name="s")


@functools.partial(
    pl.kernel,
    mesh=_mesh,
    compiler_params=pltpu.CompilerParams(use_tc_tiling_on_sc=False),
    out_type=jax.ShapeDtypeStruct((B, D_MODEL), jnp.float32),
    scratch_types=[
        pltpu.VMEM((NCHUNK, CHUNK), jnp.int32),
        pltpu.VMEM((CHUNK, D_MODEL), jnp.float32),
        pltpu.SemaphoreType.DMA,
    ],
)
def _emb_lookup(x_hbm, lut_hbm, out_hbm, idx_v, rows_v, sem):
    wid = lax.axis_index("s") * NUM_CORES + lax.axis_index("c")
    base = wid * NCHUNK

    pltpu.sync_copy(x_hbm.at[pl.ds(base, NCHUNK)], idx_v)

    def chunk_body(j, carry):
        pltpu.async_copy(lut_hbm.at[idx_v.at[j]], rows_v, sem).wait()

        def scale_body(r, c2):
            for c in range(D_MODEL // 16):
                sl = pl.ds(c * 16, 16)
                rows_v[r, sl] = rows_v[r, sl] * SCALE
            return c2

        lax.fori_loop(0, CHUNK, scale_body, 0)
        pltpu.sync_copy(rows_v, out_hbm.at[pl.ds((base + j) * CHUNK, CHUNK)])
        return carry

    lax.fori_loop(0, NCHUNK, chunk_body, 0)


def kernel(x, lut):
    x_r = x.reshape(NW * NCHUNK, CHUNK).astype(jnp.int32)
    out = _emb_lookup(x_r, lut)
    return out.reshape(ROWS, COLS, D_MODEL)


# speedup vs baseline: 1.2113x; 1.2113x over previous
import functools
import math

import jax
import jax.numpy as jnp
from jax import lax
from jax.experimental import pallas as pl
from jax.experimental.pallas import tpu as pltpu
from jax.experimental.pallas import tpu_sc as plsc

D_MODEL = 64
ROWS = 4096
COLS = 200
B = ROWS * COLS
NUM_CORES = 2
NUM_SUBCORES = 16
NW = NUM_CORES * NUM_SUBCORES
CHUNK = 128
NCHUNK = B // (NW * CHUNK)
NBUF = 4
SCALE = math.sqrt(D_MODEL)

_mesh = plsc.VectorSubcoreMesh(core_axis_name="c", subcore_axis_name="s")


@functools.partial(
    pl.kernel,
    mesh=_mesh,
    compiler_params=pltpu.CompilerParams(use_tc_tiling_on_sc=False),
    out_type=jax.ShapeDtypeStruct((B, D_MODEL), jnp.float32),
    scratch_types=[
        pltpu.VMEM((NCHUNK, CHUNK), jnp.int32),
        pltpu.VMEM((NBUF, CHUNK, D_MODEL), jnp.float32),
        pltpu.VMEM((NBUF, CHUNK, D_MODEL), jnp.float32),
        pltpu.SemaphoreType.DMA((NBUF,)),
        pltpu.SemaphoreType.DMA((NBUF,)),
    ],
)
def _emb_lookup(x_hbm, lut_hbm, out_hbm, idx_v, gbuf, sbuf, gsem, ssem):
    wid = lax.axis_index("s") * NUM_CORES + lax.axis_index("c")
    base = wid * NCHUNK

    pltpu.sync_copy(x_hbm.at[pl.ds(base, NCHUNK)], idx_v)

    for b in range(NBUF):
        pltpu.async_copy(lut_hbm.at[idx_v.at[b]], gbuf.at[b], gsem.at[b])

    def round_body(it, carry):
        j0 = it * NBUF
        for b in range(NBUF):
            j = j0 + b
            pltpu.make_async_copy(
                lut_hbm.at[idx_v.at[j]], gbuf.at[b], gsem.at[b]).wait()
            @pl.when(j >= NBUF)
            def _():
                pltpu.make_async_copy(
                    sbuf.at[b],
                    out_hbm.at[pl.ds((base + j - NBUF) * CHUNK, CHUNK)],
                    ssem.at[b]).wait()

            def scale_body(r4, c2):
                r0 = r4 * 4
                for dr in range(4):
                    for c in range(D_MODEL // 16):
                        sl = pl.ds(c * 16, 16)
                        sbuf[b, r0 + dr, sl] = gbuf[b, r0 + dr, sl] * SCALE
                return c2

            lax.fori_loop(0, CHUNK // 4, scale_body, 0)

            @pl.when(j + NBUF < NCHUNK)
            def _():
                pltpu.async_copy(
                    lut_hbm.at[idx_v.at[j + NBUF]], gbuf.at[b], gsem.at[b])

            pltpu.async_copy(
                sbuf.at[b],
                out_hbm.at[pl.ds((base + j) * CHUNK, CHUNK)], ssem.at[b])
        return carry

    lax.fori_loop(0, NCHUNK // NBUF, round_body, 0)

    for b in range(NBUF):
        j = NCHUNK - NBUF + b
        pltpu.make_async_copy(
            sbuf.at[b],
            out_hbm.at[pl.ds((base + j) * CHUNK, CHUNK)], ssem.at[b]).wait()


def kernel(x, lut):
    x_r = x.reshape(NW * NCHUNK, CHUNK).astype(jnp.int32)
    out = _emb_lookup(x_r, lut)
    return out.reshape(ROWS, COLS, D_MODEL)
